# Initial kernel scaffold; baseline (speedup 1.0000x reference)
import jax
import jax.numpy as jnp
from jax.experimental import pallas as pl


def kernel(near, far, density):
    raise NotImplementedError("write your pallas kernel here")


# speedup vs baseline: 1365.7935x; 1365.7935x over previous
import functools

import jax
import jax.numpy as jnp
from jax import lax
from jax.experimental import pallas as pl
from jax.experimental.pallas import tpu as pltpu
from jax.experimental.pallas import tpu_sc as plsc

N = 64
NF = 128
NO = N + NF
NC, NS, L = 2, 16, 16
NW = NC * NS

_S = float(0.9 / 127.0)
_INV_S = float(127.0 / 0.9)


def _sampler_body(RW, G, near_hbm, far_hbm, dens_hbm, out_hbm,
                  near_v, far_v, dens_v, out_v, ebuf, atab, btab, cnt):
    wid = lax.axis_index("c") * NS + lax.axis_index("s")
    base = wid * RW
    pltpu.sync_copy(near_hbm.at[pl.ds(base, RW)], near_v)
    pltpu.sync_copy(far_hbm.at[pl.ds(base, RW)], far_v)
    pltpu.sync_copy(dens_hbm.at[pl.ds(base * N, RW * N)], dens_v)

    lane = lax.iota(jnp.int32, L)
    zeros_f = jnp.zeros((L,), jnp.float32)
    zeros_i = jnp.zeros((L,), jnp.int32)
    ones_i = jnp.ones((L,), jnp.int32)

    def zinit(j, carry):
        cnt[pl.ds(j * L, L)] = zeros_i
        return carry

    lax.fori_loop(0, NF + 1, zinit, 0)

    def group(g, carry):
        gl = g * L
        nearv = near_v[pl.ds(gl, L)]
        farv = far_v[pl.ds(gl, L)]
        h = (farv - nearv) * (1.0 / N)
        dbase = g * (L * N) + lane * N
        obase = g * (L * NO) + lane * NO

        def p1(i, S):
            d = plsc.load_gather(dens_v, [dbase + i])
            S = S + d
            ebuf[pl.ds(i * L, L)] = jnp.exp(-h * S)
            return S

        lax.fori_loop(0, N, p1, zeros_f)
        e_last = ebuf[pl.ds((N - 1) * L, L)]
        inv_w = 1.0 / ((1.0 - e_last) + 1e-6)

        def p2(i, carry):
            prev_cdf, prev_mid = carry
            e = ebuf[pl.ds(i * L, L)]
            cdf = (1.0 - e) * inv_w
            midv = nearv + (i.astype(jnp.float32) + 0.5) * h
            x = (cdf - 0.05) * _INV_S
            t = x.astype(jnp.int32)
            c = jnp.where(t.astype(jnp.float32) < x, t + 1, t)
            f = jnp.clip(c, 0, NF)
            plsc.addupdate_scatter(cnt, [f * L + lane], ones_i)
            plsc.store_scatter(out_v, [obase + i + f], midv)
            dn = cdf - prev_cdf
            dnw = jnp.where(dn < 1e-5, jnp.float32(1.0), dn)
            bk = h / dnw
            atab[pl.ds(i * L, L)] = prev_mid - prev_cdf * bk
            btab[pl.ds(i * L, L)] = bk
            return (cdf, midv)

        _, mid_last = lax.fori_loop(0, N, p2, (zeros_f, nearv))
        atab[pl.ds(0, L)] = nearv + 0.5 * h
        btab[pl.ds(0, L)] = zeros_f
        atab[pl.ds(N * L, L)] = mid_last
        btab[pl.ds(N * L, L)] = zeros_f

        def p3(j, run):
            run = run + cnt[pl.ds(j * L, L)]
            cnt[pl.ds(j * L, L)] = zeros_i
            av = plsc.load_gather(atab, [run * L + lane])
            bv = plsc.load_gather(btab, [run * L + lane])
            u = 0.05 + j.astype(jnp.float32) * _S
            plsc.store_scatter(out_v, [obase + j + run], av + bv * u)
            return run

        lax.fori_loop(0, NF, p3, zeros_i)
        cnt[pl.ds(NF * L, L)] = zeros_i
        return carry

    lax.fori_loop(0, G, group, 0)
    pltpu.sync_copy(out_v, out_hbm.at[pl.ds(base * NO, RW * NO)])


def kernel(near, far, density):
    B = density.shape[0]
    RW = B // NW
    G = RW // L

    mesh = plsc.VectorSubcoreMesh(
        core_axis_name="c", subcore_axis_name="s",
        num_cores=NC, num_subcores=NS)

    fn = pl.kernel(
        functools.partial(_sampler_body, RW, G),
        out_type=jax.ShapeDtypeStruct((B * NO,), jnp.float32),
        mesh=mesh,
        compiler_params=pltpu.CompilerParams(
            needs_layout_passes=False, use_tc_tiling_on_sc=False),
        scratch_types=[
            pltpu.VMEM((RW,), jnp.float32),
            pltpu.VMEM((RW,), jnp.float32),
            pltpu.VMEM((RW * N,), jnp.float32),
            pltpu.VMEM((RW * NO,), jnp.float32),
            pltpu.VMEM((N * L,), jnp.float32),
            pltpu.VMEM(((N + 1) * L,), jnp.float32),
            pltpu.VMEM(((N + 1) * L,), jnp.float32),
            pltpu.VMEM(((NF + 1) * L,), jnp.int32),
        ],
    )
    out = fn(near.reshape(B), far.reshape(B), density.reshape(B * N))
    return out.reshape(B, NO)


# speedup vs baseline: 1382.7643x; 1.0124x over previous
import functools

import jax
import jax.numpy as jnp
from jax import lax
from jax.experimental import pallas as pl
from jax.experimental.pallas import tpu as pltpu
from jax.experimental.pallas import tpu_sc as plsc

N = 64
NF = 128
NO = N + NF
NC, NS, L = 2, 16, 16
NW = NC * NS
U = 4

_S = float(0.9 / 127.0)
_INV_S = float(127.0 / 0.9)


def _sampler_body(RW, G, near_hbm, far_hbm, dens_hbm, out_hbm,
                  near_v, far_v, dens_v, out_v, ebuf, atab, btab, cnt):
    wid = lax.axis_index("c") * NS + lax.axis_index("s")
    base = wid * RW
    pltpu.sync_copy(near_hbm.at[pl.ds(base, RW)], near_v)
    pltpu.sync_copy(far_hbm.at[pl.ds(base, RW)], far_v)
    pltpu.sync_copy(dens_hbm.at[pl.ds(base * N, RW * N)], dens_v)

    lane = lax.iota(jnp.int32, L)
    zeros_f = jnp.zeros((L,), jnp.float32)
    zeros_i = jnp.zeros((L,), jnp.int32)
    ones_i = jnp.ones((L,), jnp.int32)

    def zinit(j, carry):
        for k in range(U):
            cnt[pl.ds((k * (NF + 1) + j) * L, L)] = zeros_i
        return carry

    lax.fori_loop(0, NF + 1, zinit, 0)

    def chunk(gc, carry):
        g0 = gc * U
        nearv = [near_v[pl.ds((g0 + k) * L, L)] for k in range(U)]
        farv = [far_v[pl.ds((g0 + k) * L, L)] for k in range(U)]
        h = [(farv[k] - nearv[k]) * (1.0 / N) for k in range(U)]
        dbase = [(g0 + k) * (L * N) + lane * N for k in range(U)]
        obase = [(g0 + k) * (L * NO) + lane * NO for k in range(U)]

        def p1(i, S):
            out = []
            for k in range(U):
                d = plsc.load_gather(dens_v, [dbase[k] + i])
                s = S[k] + d
                ebuf[pl.ds((k * N) * L + i * L, L)] = jnp.exp(-h[k] * s)
                out.append(s)
            return tuple(out)

        lax.fori_loop(0, N, p1, (zeros_f,) * U)
        inv_w = []
        for k in range(U):
            e_last = ebuf[pl.ds((k * N + N - 1) * L, L)]
            inv_w.append(1.0 / ((1.0 - e_last) + 1e-6))

        def p2(i, carry):
            res = []
            for k in range(U):
                prev_cdf, prev_mid = carry[2 * k], carry[2 * k + 1]
                e = ebuf[pl.ds((k * N) * L + i * L, L)]
                cdf = (1.0 - e) * inv_w[k]
                midv = nearv[k] + (i.astype(jnp.float32) + 0.5) * h[k]
                x = (cdf - 0.05) * _INV_S
                t = x.astype(jnp.int32)
                c = jnp.where(t.astype(jnp.float32) < x, t + 1, t)
                f = jnp.clip(c, 0, NF)
                plsc.addupdate_scatter(
                    cnt, [(k * (NF + 1)) * L + f * L + lane], ones_i)
                plsc.store_scatter(out_v, [obase[k] + i + f], midv)
                dn = cdf - prev_cdf
                dnw = jnp.where(dn < 1e-5, jnp.float32(1.0), dn)
                bk = h[k] / dnw
                atab[pl.ds((k * (N + 1)) * L + i * L, L)] = (
                    prev_mid - prev_cdf * bk)
                btab[pl.ds((k * (N + 1)) * L + i * L, L)] = bk
                res += [cdf, midv]
            return tuple(res)

        init = ()
        for k in range(U):
            init += (zeros_f, nearv[k])
        fin = lax.fori_loop(0, N, p2, init)
        for k in range(U):
            atab[pl.ds((k * (N + 1)) * L, L)] = nearv[k] + 0.5 * h[k]
            btab[pl.ds((k * (N + 1)) * L, L)] = zeros_f
            atab[pl.ds((k * (N + 1) + N) * L, L)] = fin[2 * k + 1]
            btab[pl.ds((k * (N + 1) + N) * L, L)] = zeros_f

        def p3(j, run):
            out = []
            u = 0.05 + j.astype(jnp.float32) * _S
            for k in range(U):
                r = run[k] + cnt[pl.ds((k * (NF + 1)) * L + j * L, L)]
                cnt[pl.ds((k * (NF + 1)) * L + j * L, L)] = zeros_i
                av = plsc.load_gather(atab, [(k * (N + 1)) * L + r * L + lane])
                bv = plsc.load_gather(btab, [(k * (N + 1)) * L + r * L + lane])
                plsc.store_scatter(out_v, [obase[k] + j + r], av + bv * u)
                out.append(r)
            return tuple(out)

        lax.fori_loop(0, NF, p3, (zeros_i,) * U)
        for k in range(U):
            cnt[pl.ds((k * (NF + 1) + NF) * L, L)] = zeros_i
        return carry

    lax.fori_loop(0, G // U, chunk, 0)
    pltpu.sync_copy(out_v, out_hbm.at[pl.ds(base * NO, RW * NO)])


def kernel(near, far, density):
    B = density.shape[0]
    RW = B // NW
    G = RW // L

    mesh = plsc.VectorSubcoreMesh(
        core_axis_name="c", subcore_axis_name="s",
        num_cores=NC, num_subcores=NS)

    fn = pl.kernel(
        functools.partial(_sampler_body, RW, G),
        out_type=jax.ShapeDtypeStruct((B * NO,), jnp.float32),
        mesh=mesh,
        compiler_params=pltpu.CompilerParams(
            needs_layout_passes=False, use_tc_tiling_on_sc=False),
        scratch_types=[
            pltpu.VMEM((RW,), jnp.float32),
            pltpu.VMEM((RW,), jnp.float32),
            pltpu.VMEM((RW * N,), jnp.float32),
            pltpu.VMEM((RW * NO,), jnp.float32),
            pltpu.VMEM((U * N * L,), jnp.float32),
            pltpu.VMEM((U * (N + 1) * L,), jnp.float32),
            pltpu.VMEM((U * (N + 1) * L,), jnp.float32),
            pltpu.VMEM((U * (NF + 1) * L,), jnp.int32),
        ],
    )
    out = fn(near.reshape(B), far.reshape(B), density.reshape(B * N))
    return out.reshape(B, NO)


# speedup vs baseline: 1479.5520x; 1.0700x over previous
import functools

import jax
import jax.numpy as jnp
from jax import lax
from jax.experimental import pallas as pl
from jax.experimental.pallas import tpu as pltpu
from jax.experimental.pallas import tpu_sc as plsc

N = 64
NF = 128
NO = N + NF
NP = N + 1
NOP = NO + 1
NC, NS, L = 2, 16, 16
NW = NC * NS
U = 4

_S = float(0.9 / 127.0)
_INV_S = float(127.0 / 0.9)


def _sampler_body(RW, G, near_hbm, far_hbm, dens_hbm, out_hbm,
                  near_v, far_v, dens_v, out_v, ebuf, atab, btab, cnt):
    wid = lax.axis_index("c") * NS + lax.axis_index("s")
    base = wid * RW
    pltpu.sync_copy(near_hbm.at[pl.ds(base, RW)], near_v)
    pltpu.sync_copy(far_hbm.at[pl.ds(base, RW)], far_v)
    pltpu.sync_copy(dens_hbm.at[pl.ds(base, RW), :], dens_v.at[:, 0:N])

    lane = lax.iota(jnp.int32, L)
    zeros_f = jnp.zeros((L,), jnp.float32)
    zeros_i = jnp.zeros((L,), jnp.int32)
    ones_i = jnp.ones((L,), jnp.int32)

    def zinit(j, carry):
        for k in range(U):
            cnt[pl.ds((k * (NF + 1) + j) * L, L)] = zeros_i
        return carry

    lax.fori_loop(0, NF + 1, zinit, 0)

    def chunk(gc, carry):
        g0 = gc * U
        nearv = [near_v[pl.ds((g0 + k) * L, L)] for k in range(U)]
        farv = [far_v[pl.ds((g0 + k) * L, L)] for k in range(U)]
        h = [(farv[k] - nearv[k]) * (1.0 / N) for k in range(U)]
        rows = [(g0 + k) * L + lane for k in range(U)]

        def p1(i, S):
            out = []
            for k in range(U):
                d = plsc.load_gather(dens_v, [rows[k], jnp.full((L,), i, jnp.int32)])
                s = S[k] + d
                ebuf[pl.ds((k * N) * L + i * L, L)] = jnp.exp(-h[k] * s)
                out.append(s)
            return tuple(out)

        lax.fori_loop(0, N, p1, (zeros_f,) * U)
        inv_w = []
        for k in range(U):
            e_last = ebuf[pl.ds((k * N + N - 1) * L, L)]
            inv_w.append(1.0 / ((1.0 - e_last) + 1e-6))

        def p2(i, carry):
            res = []
            for k in range(U):
                prev_cdf, prev_mid = carry[2 * k], carry[2 * k + 1]
                e = ebuf[pl.ds((k * N) * L + i * L, L)]
                cdf = (1.0 - e) * inv_w[k]
                midv = nearv[k] + (i.astype(jnp.float32) + 0.5) * h[k]
                x = (cdf - 0.05) * _INV_S
                t = x.astype(jnp.int32)
                c = jnp.where(t.astype(jnp.float32) < x, t + 1, t)
                f = jnp.clip(c, 0, NF)
                plsc.addupdate_scatter(
                    cnt, [(k * (NF + 1)) * L + f * L + lane], ones_i)
                plsc.store_scatter(out_v, [rows[k], i + f], midv)
                dn = cdf - prev_cdf
                dnw = jnp.where(dn < 1e-5, jnp.float32(1.0), dn)
                bk = h[k] / dnw
                atab[pl.ds((k * (N + 1)) * L + i * L, L)] = (
                    prev_mid - prev_cdf * bk)
                btab[pl.ds((k * (N + 1)) * L + i * L, L)] = bk
                res += [cdf, midv]
            return tuple(res)

        init = ()
        for k in range(U):
            init += (zeros_f, nearv[k])
        fin = lax.fori_loop(0, N, p2, init)
        for k in range(U):
            atab[pl.ds((k * (N + 1)) * L, L)] = nearv[k] + 0.5 * h[k]
            btab[pl.ds((k * (N + 1)) * L, L)] = zeros_f
            atab[pl.ds((k * (N + 1) + N) * L, L)] = fin[2 * k + 1]
            btab[pl.ds((k * (N + 1) + N) * L, L)] = zeros_f

        def p3(j, run):
            out = []
            u = 0.05 + j.astype(jnp.float32) * _S
            for k in range(U):
                r = run[k] + cnt[pl.ds((k * (NF + 1)) * L + j * L, L)]
                cnt[pl.ds((k * (NF + 1)) * L + j * L, L)] = zeros_i
                av = plsc.load_gather(atab, [(k * (N + 1)) * L + r * L + lane])
                bv = plsc.load_gather(btab, [(k * (N + 1)) * L + r * L + lane])
                plsc.store_scatter(out_v, [rows[k], j + r], av + bv * u)
                out.append(r)
            return tuple(out)

        lax.fori_loop(0, NF, p3, (zeros_i,) * U)
        for k in range(U):
            cnt[pl.ds((k * (NF + 1) + NF) * L, L)] = zeros_i
        return carry

    lax.fori_loop(0, G // U, chunk, 0)
    pltpu.sync_copy(out_v.at[:, 0:NO], out_hbm.at[pl.ds(base, RW), :])


def kernel(near, far, density):
    B = density.shape[0]
    RW = B // NW
    G = RW // L

    mesh = plsc.VectorSubcoreMesh(
        core_axis_name="c", subcore_axis_name="s",
        num_cores=NC, num_subcores=NS)

    fn = pl.kernel(
        functools.partial(_sampler_body, RW, G),
        out_type=jax.ShapeDtypeStruct((B, NO), jnp.float32),
        mesh=mesh,
        compiler_params=pltpu.CompilerParams(
            needs_layout_passes=False, use_tc_tiling_on_sc=False),
        scratch_types=[
            pltpu.VMEM((RW,), jnp.float32),
            pltpu.VMEM((RW,), jnp.float32),
            pltpu.VMEM((RW, NP), jnp.float32),
            pltpu.VMEM((RW, NOP), jnp.float32),
            pltpu.VMEM((U * N * L,), jnp.float32),
            pltpu.VMEM((U * (N + 1) * L,), jnp.float32),
            pltpu.VMEM((U * (N + 1) * L,), jnp.float32),
            pltpu.VMEM((U * (NF + 1) * L,), jnp.int32),
        ],
    )
    return fn(near.reshape(B), far.reshape(B), density)


# speedup vs baseline: 2614.0437x; 1.7668x over previous
import functools

import jax
import jax.numpy as jnp
from jax import lax
from jax.experimental import pallas as pl
from jax.experimental.pallas import tpu as pltpu
from jax.experimental.pallas import tpu_sc as plsc

N = 64
NF = 128
NO = N + NF
NP = N + 1
NOP = NO + 1
NC, NS, L = 2, 16, 16
NW = NC * NS
U = 4

_S = float(0.9 / 127.0)
_INV_S = float(127.0 / 0.9)


def _sampler_body(RW, G, near_hbm, far_hbm, dens_hbm, out_hbm,
                  near_v, far_v, dens_v, out_v, ebuf, atab, btab, cnt):
    wid = lax.axis_index("c") * NS + lax.axis_index("s")
    base = wid * RW
    pltpu.sync_copy(near_hbm.at[pl.ds(base, RW)], near_v)
    pltpu.sync_copy(far_hbm.at[pl.ds(base, RW)], far_v)
    pltpu.sync_copy(dens_hbm.at[pl.ds(base, RW), :], dens_v.at[:, 0:N])

    lane = lax.iota(jnp.int32, L)
    zeros_f = jnp.zeros((L,), jnp.float32)
    zeros_i = jnp.zeros((L,), jnp.int32)
    ones_i = jnp.ones((L,), jnp.int32)

    def zinit(j, carry):
        for k in range(U):
            cnt[pl.ds((k * (NF + 1) + j) * L, L)] = zeros_i
        return carry

    lax.fori_loop(0, NF + 1, zinit, 0)

    def chunk(gc, carry):
        g0 = gc * U
        nearv = [near_v[pl.ds((g0 + k) * L, L)] for k in range(U)]
        farv = [far_v[pl.ds((g0 + k) * L, L)] for k in range(U)]
        h = [(farv[k] - nearv[k]) * (1.0 / N) for k in range(U)]
        rows = [(g0 + k) * L + lane for k in range(U)]

        @plsc.parallel_loop(0, N, carry=(zeros_f,) * U)
        def _p1(i, S):
            out = []
            for k in range(U):
                d = plsc.load_gather(dens_v, [rows[k], jnp.full((L,), i, jnp.int32)])
                s = S[k] + d
                ebuf[pl.ds((k * N) * L + i * L, L)] = jnp.exp(-h[k] * s)
                out.append(s)
            return tuple(out)
        inv_w = []
        for k in range(U):
            e_last = ebuf[pl.ds((k * N + N - 1) * L, L)]
            inv_w.append(1.0 / ((1.0 - e_last) + 1e-6))

        init = ()
        for k in range(U):
            init += (zeros_f, nearv[k])

        @plsc.parallel_loop(0, N, carry=init)
        def fin(i, carry):
            res = []
            for k in range(U):
                prev_cdf, prev_mid = carry[2 * k], carry[2 * k + 1]
                e = ebuf[pl.ds((k * N) * L + i * L, L)]
                cdf = (1.0 - e) * inv_w[k]
                midv = nearv[k] + (i.astype(jnp.float32) + 0.5) * h[k]
                x = (cdf - 0.05) * _INV_S
                t = x.astype(jnp.int32)
                c = jnp.where(t.astype(jnp.float32) < x, t + 1, t)
                f = jnp.clip(c, 0, NF)
                plsc.addupdate_scatter(
                    cnt, [(k * (NF + 1)) * L + f * L + lane], ones_i)
                plsc.store_scatter(out_v, [rows[k], i + f], midv)
                dn = cdf - prev_cdf
                dnw = jnp.where(dn < 1e-5, jnp.float32(1.0), dn)
                bk = h[k] / dnw
                atab[pl.ds((k * (N + 1)) * L + i * L, L)] = (
                    prev_mid - prev_cdf * bk)
                btab[pl.ds((k * (N + 1)) * L + i * L, L)] = bk
                res += [cdf, midv]
            return tuple(res)

        for k in range(U):
            atab[pl.ds((k * (N + 1)) * L, L)] = nearv[k] + 0.5 * h[k]
            btab[pl.ds((k * (N + 1)) * L, L)] = zeros_f
            atab[pl.ds((k * (N + 1) + N) * L, L)] = fin[2 * k + 1]
            btab[pl.ds((k * (N + 1) + N) * L, L)] = zeros_f

        @plsc.parallel_loop(0, NF, carry=(zeros_i,) * U)
        def _p3(j, run):
            out = []
            u = 0.05 + j.astype(jnp.float32) * _S
            for k in range(U):
                r = run[k] + cnt[pl.ds((k * (NF + 1)) * L + j * L, L)]
                cnt[pl.ds((k * (NF + 1)) * L + j * L, L)] = zeros_i
                av = plsc.load_gather(atab, [(k * (N + 1)) * L + r * L + lane])
                bv = plsc.load_gather(btab, [(k * (N + 1)) * L + r * L + lane])
                plsc.store_scatter(out_v, [rows[k], j + r], av + bv * u)
                out.append(r)
            return tuple(out)
        for k in range(U):
            cnt[pl.ds((k * (NF + 1) + NF) * L, L)] = zeros_i
        return carry

    lax.fori_loop(0, G // U, chunk, 0)
    pltpu.sync_copy(out_v.at[:, 0:NO], out_hbm.at[pl.ds(base, RW), :])


def kernel(near, far, density):
    B = density.shape[0]
    RW = B // NW
    G = RW // L

    mesh = plsc.VectorSubcoreMesh(
        core_axis_name="c", subcore_axis_name="s",
        num_cores=NC, num_subcores=NS)

    fn = pl.kernel(
        functools.partial(_sampler_body, RW, G),
        out_type=jax.ShapeDtypeStruct((B, NO), jnp.float32),
        mesh=mesh,
        compiler_params=pltpu.CompilerParams(
            needs_layout_passes=False, use_tc_tiling_on_sc=False),
        scratch_types=[
            pltpu.VMEM((RW,), jnp.float32),
            pltpu.VMEM((RW,), jnp.float32),
            pltpu.VMEM((RW, NP), jnp.float32),
            pltpu.VMEM((RW, NOP), jnp.float32),
            pltpu.VMEM((U * N * L,), jnp.float32),
            pltpu.VMEM((U * (N + 1) * L,), jnp.float32),
            pltpu.VMEM((U * (N + 1) * L,), jnp.float32),
            pltpu.VMEM((U * (NF + 1) * L,), jnp.int32),
        ],
    )
    return fn(near.reshape(B), far.reshape(B), density)


# speedup vs baseline: 2683.7079x; 1.0266x over previous
import functools

import jax
import jax.numpy as jnp
from jax import lax
from jax.experimental import pallas as pl
from jax.experimental.pallas import tpu as pltpu
from jax.experimental.pallas import tpu_sc as plsc

N = 64
NF = 128
NO = N + NF
NP = N + 1
NOP = NO + 1
NC, NS, L = 2, 16, 16
NW = NC * NS
U = 4

_S = float(0.9 / 127.0)
_INV_S = float(127.0 / 0.9)


def _sampler_body(RW, G, near_hbm, far_hbm, dens_hbm, out_hbm,
                  near_v, far_v, dens_v, out_v, ebuf, atab, btab, cnt):
    wid = lax.axis_index("c") * NS + lax.axis_index("s")
    base = wid * RW
    pltpu.sync_copy(near_hbm.at[pl.ds(base, RW)], near_v)
    pltpu.sync_copy(far_hbm.at[pl.ds(base, RW)], far_v)
    pltpu.sync_copy(dens_hbm.at[pl.ds(base, RW), :], dens_v.at[:, 0:N])

    lane = lax.iota(jnp.int32, L)
    zeros_f = jnp.zeros((L,), jnp.float32)
    zeros_i = jnp.zeros((L,), jnp.int32)
    ones_i = jnp.ones((L,), jnp.int32)

    def zinit(j, carry):
        for k in range(U):
            cnt[pl.ds((k * (NF + 1) + j) * L, L)] = zeros_i
        return carry

    lax.fori_loop(0, NF + 1, zinit, 0)

    def chunk(gc, carry):
        g0 = gc * U
        nearv = [near_v[pl.ds((g0 + k) * L, L)] for k in range(U)]
        farv = [far_v[pl.ds((g0 + k) * L, L)] for k in range(U)]
        h = [(farv[k] - nearv[k]) * (1.0 / N) for k in range(U)]
        rows = [(g0 + k) * L + lane for k in range(U)]

        @plsc.parallel_loop(0, N, unroll=2, carry=(zeros_f,) * U)
        def _p1(i, S):
            out = []
            for k in range(U):
                d = plsc.load_gather(dens_v, [rows[k], jnp.full((L,), i, jnp.int32)])
                s = S[k] + d
                ebuf[pl.ds((k * N) * L + i * L, L)] = jnp.exp(-h[k] * s)
                out.append(s)
            return tuple(out)
        inv_w = []
        for k in range(U):
            e_last = ebuf[pl.ds((k * N + N - 1) * L, L)]
            inv_w.append(1.0 / ((1.0 - e_last) + 1e-6))

        init = ()
        for k in range(U):
            init += (zeros_f, nearv[k])

        @plsc.parallel_loop(0, N, unroll=2, carry=init)
        def fin(i, carry):
            res = []
            for k in range(U):
                prev_cdf, prev_mid = carry[2 * k], carry[2 * k + 1]
                e = ebuf[pl.ds((k * N) * L + i * L, L)]
                cdf = (1.0 - e) * inv_w[k]
                midv = nearv[k] + (i.astype(jnp.float32) + 0.5) * h[k]
                x = (cdf - 0.05) * _INV_S
                t = x.astype(jnp.int32)
                c = jnp.where(t.astype(jnp.float32) < x, t + 1, t)
                f = jnp.clip(c, 0, NF)
                plsc.addupdate_scatter(
                    cnt, [(k * (NF + 1)) * L + f * L + lane], ones_i)
                plsc.store_scatter(out_v, [rows[k], i + f], midv)
                dn = cdf - prev_cdf
                dnw = jnp.where(dn < 1e-5, jnp.float32(1.0), dn)
                bk = h[k] / dnw
                atab[pl.ds((k * (N + 1)) * L + i * L, L)] = (
                    prev_mid - prev_cdf * bk)
                btab[pl.ds((k * (N + 1)) * L + i * L, L)] = bk
                res += [cdf, midv]
            return tuple(res)

        for k in range(U):
            atab[pl.ds((k * (N + 1)) * L, L)] = nearv[k] + 0.5 * h[k]
            btab[pl.ds((k * (N + 1)) * L, L)] = zeros_f
            atab[pl.ds((k * (N + 1) + N) * L, L)] = fin[2 * k + 1]
            btab[pl.ds((k * (N + 1) + N) * L, L)] = zeros_f

        @plsc.parallel_loop(0, NF, unroll=2, carry=(zeros_i,) * U)
        def _p3(j, run):
            out = []
            u = 0.05 + j.astype(jnp.float32) * _S
            for k in range(U):
                r = run[k] + cnt[pl.ds((k * (NF + 1)) * L + j * L, L)]
                cnt[pl.ds((k * (NF + 1)) * L + j * L, L)] = zeros_i
                av = plsc.load_gather(atab, [(k * (N + 1)) * L + r * L + lane])
                bv = plsc.load_gather(btab, [(k * (N + 1)) * L + r * L + lane])
                plsc.store_scatter(out_v, [rows[k], j + r], av + bv * u)
                out.append(r)
            return tuple(out)
        for k in range(U):
            cnt[pl.ds((k * (NF + 1) + NF) * L, L)] = zeros_i
        return carry

    lax.fori_loop(0, G // U, chunk, 0)
    pltpu.sync_copy(out_v.at[:, 0:NO], out_hbm.at[pl.ds(base, RW), :])


def kernel(near, far, density):
    B = density.shape[0]
    RW = B // NW
    G = RW // L

    mesh = plsc.VectorSubcoreMesh(
        core_axis_name="c", subcore_axis_name="s",
        num_cores=NC, num_subcores=NS)

    fn = pl.kernel(
        functools.partial(_sampler_body, RW, G),
        out_type=jax.ShapeDtypeStruct((B, NO), jnp.float32),
        mesh=mesh,
        compiler_params=pltpu.CompilerParams(
            needs_layout_passes=False, use_tc_tiling_on_sc=False),
        scratch_types=[
            pltpu.VMEM((RW,), jnp.float32),
            pltpu.VMEM((RW,), jnp.float32),
            pltpu.VMEM((RW, NP), jnp.float32),
            pltpu.VMEM((RW, NOP), jnp.float32),
            pltpu.VMEM((U * N * L,), jnp.float32),
            pltpu.VMEM((U * (N + 1) * L,), jnp.float32),
            pltpu.VMEM((U * (N + 1) * L,), jnp.float32),
            pltpu.VMEM((U * (NF + 1) * L,), jnp.int32),
        ],
    )
    return fn(near.reshape(B), far.reshape(B), density)


# speedup vs baseline: 3767.0124x; 1.4037x over previous
import functools

import jax
import jax.numpy as jnp
from jax import lax
from jax.experimental import pallas as pl
from jax.experimental.pallas import tpu as pltpu
from jax.experimental.pallas import tpu_sc as plsc

N = 64
NF = 128
NO = N + NF
NP = N + 1
NOP = NO + 1
NC, NS, L = 2, 16, 16
NW = NC * NS
U = 4

_S = float(0.9 / 127.0)
_INV_S = float(127.0 / 0.9)


def _sampler_body(RW, G, near_hbm, far_hbm, dens_hbm, out_hbm,
                  near_v, far_v, dens_v, out_v, ebuf, atab, btab, cnt):
    wid = lax.axis_index("c") * NS + lax.axis_index("s")
    base = wid * RW
    pltpu.sync_copy(near_hbm.at[pl.ds(base, RW)], near_v)
    pltpu.sync_copy(far_hbm.at[pl.ds(base, RW)], far_v)
    pltpu.sync_copy(dens_hbm.at[pl.ds(base, RW), :], dens_v.at[:, 0:N])

    lane = lax.iota(jnp.int32, L)
    zeros_f = jnp.zeros((L,), jnp.float32)
    zeros_i = jnp.zeros((L,), jnp.int32)
    ones_i = jnp.ones((L,), jnp.int32)

    def zinit(j, carry):
        for k in range(U):
            cnt[pl.ds((k * (NF + 1) + j) * L, L)] = zeros_i
        return carry

    lax.fori_loop(0, NF + 1, zinit, 0)

    def chunk(gc, carry):
        g0 = gc * U
        nearv = [near_v[pl.ds((g0 + k) * L, L)] for k in range(U)]
        farv = [far_v[pl.ds((g0 + k) * L, L)] for k in range(U)]
        h = [(farv[k] - nearv[k]) * (1.0 / N) for k in range(U)]
        rows = [(g0 + k) * L + lane for k in range(U)]
        ob = [((g0 + k) >> 3) * 1024 + ((g0 + k) & 7) * 16 + lane
              for k in range(U)]

        @plsc.parallel_loop(0, N, unroll=2, carry=(zeros_f,) * U)
        def _p1(i, S):
            out = []
            for k in range(U):
                d = plsc.load_gather(dens_v, [rows[k], jnp.full((L,), i, jnp.int32)])
                s = S[k] + d
                ebuf[pl.ds((k * N) * L + i * L, L)] = jnp.exp(-h[k] * s)
                out.append(s)
            return tuple(out)
        inv_w = []
        for k in range(U):
            e_last = ebuf[pl.ds((k * N + N - 1) * L, L)]
            inv_w.append(1.0 / ((1.0 - e_last) + 1e-6))

        init = ()
        for k in range(U):
            init += (zeros_f, nearv[k])

        @plsc.parallel_loop(0, N, unroll=2, carry=init)
        def fin(i, carry):
            res = []
            for k in range(U):
                prev_cdf, prev_mid = carry[2 * k], carry[2 * k + 1]
                e = ebuf[pl.ds((k * N) * L + i * L, L)]
                cdf = (1.0 - e) * inv_w[k]
                midv = nearv[k] + (i.astype(jnp.float32) + 0.5) * h[k]
                x = (cdf - 0.05) * _INV_S
                t = x.astype(jnp.int32)
                c = jnp.where(t.astype(jnp.float32) < x, t + 1, t)
                f = jnp.clip(c, 0, NF)
                plsc.addupdate_scatter(
                    cnt, [(k * (NF + 1)) * L + f * L + lane], ones_i)
                slot = i + f
                plsc.store_scatter(
                    out_v,
                    [slot >> 3, ((slot & 7) << 7) + ob[k]], midv)
                dn = cdf - prev_cdf
                dnw = jnp.where(dn < 1e-5, jnp.float32(1.0), dn)
                bk = h[k] / dnw
                atab[pl.ds((k * (N + 1)) * L + i * L, L)] = (
                    prev_mid - prev_cdf * bk)
                btab[pl.ds((k * (N + 1)) * L + i * L, L)] = bk
                res += [cdf, midv]
            return tuple(res)

        for k in range(U):
            atab[pl.ds((k * (N + 1)) * L, L)] = nearv[k] + 0.5 * h[k]
            btab[pl.ds((k * (N + 1)) * L, L)] = zeros_f
            atab[pl.ds((k * (N + 1) + N) * L, L)] = fin[2 * k + 1]
            btab[pl.ds((k * (N + 1) + N) * L, L)] = zeros_f

        @plsc.parallel_loop(0, NF, unroll=2, carry=(zeros_i,) * U)
        def _p3(j, run):
            out = []
            u = 0.05 + j.astype(jnp.float32) * _S
            for k in range(U):
                r = run[k] + cnt[pl.ds((k * (NF + 1)) * L + j * L, L)]
                cnt[pl.ds((k * (NF + 1)) * L + j * L, L)] = zeros_i
                av = plsc.load_gather(atab, [(k * (N + 1)) * L + r * L + lane])
                bv = plsc.load_gather(btab, [(k * (N + 1)) * L + r * L + lane])
                slot = j + r
                plsc.store_scatter(
                    out_v,
                    [slot >> 3, ((slot & 7) << 7) + ob[k]], av + bv * u)
                out.append(r)
            return tuple(out)
        for k in range(U):
            cnt[pl.ds((k * (NF + 1) + NF) * L, L)] = zeros_i
        return carry

    lax.fori_loop(0, G // U, chunk, 0)
    RB = RW // 128
    pltpu.sync_copy(out_v, out_hbm.at[:, pl.ds(wid * (RB * 1024), RB * 1024)])


def kernel(near, far, density):
    B = density.shape[0]
    RW = B // NW
    G = RW // L

    mesh = plsc.VectorSubcoreMesh(
        core_axis_name="c", subcore_axis_name="s",
        num_cores=NC, num_subcores=NS)

    RB = RW // 128
    fn = pl.kernel(
        functools.partial(_sampler_body, RW, G),
        out_type=jax.ShapeDtypeStruct((NO // 8, (B // 128) * 1024), jnp.float32),
        mesh=mesh,
        compiler_params=pltpu.CompilerParams(
            needs_layout_passes=False, use_tc_tiling_on_sc=False),
        scratch_types=[
            pltpu.VMEM((RW,), jnp.float32),
            pltpu.VMEM((RW,), jnp.float32),
            pltpu.VMEM((RW, NP), jnp.float32),
            pltpu.VMEM((NO // 8, RB * 1024), jnp.float32),
            pltpu.VMEM((U * N * L,), jnp.float32),
            pltpu.VMEM((U * (N + 1) * L,), jnp.float32),
            pltpu.VMEM((U * (N + 1) * L,), jnp.float32),
            pltpu.VMEM((U * (NF + 1) * L,), jnp.int32),
        ],
    )
    y = fn(near.reshape(B), far.reshape(B), density)
    return (y.reshape(NO // 8, B // 128, 8, 128)
            .transpose(1, 3, 0, 2).reshape(B, NO))
